# baseline (speedup 1.0000x reference)
import functools

import jax
import jax.numpy as jnp
from jax import lax
from jax.experimental import pallas as pl
from jax.experimental.pallas import tpu as pltpu
from jax.experimental.pallas import tpu_sc as plsc

N = 10000
E = 320000
H = 128
OUT = 64
G = 64

ACCW = 136
NC = 2
NS = 16
NW = NC * NS
EPW = E // NW
BLK = 48
NBLK = 210
BN = 1000


def _edge_pass(hl, hr, epack, wa, zeros):
    mesh = plsc.VectorSubcoreMesh(core_axis_name="c", subcore_axis_name="s")

    @functools.partial(
        pl.kernel,
        mesh=mesh,
        compiler_params=pltpu.CompilerParams(
            needs_layout_passes=False, use_tc_tiling_on_sc=False),
        out_type=jax.ShapeDtypeStruct((NC, N, ACCW), jnp.float32),
        scratch_types=[
            pltpu.VMEM((2, 3, BLK), jnp.int32),
            pltpu.VMEM((2, BLK, H), jnp.float32),
            pltpu.VMEM((2, BLK, H), jnp.float32),
            pltpu.VMEM((2, BLK, ACCW), jnp.float32),
            pltpu.VMEM((2, BLK), jnp.int32),
            pltpu.VMEM((2, BLK), jnp.int32),
            pltpu.VMEM((2, H), jnp.float32),
            pltpu.VMEM_SHARED((N, ACCW), jnp.float32),
            pltpu.SemaphoreType.DMA,
            pltpu.SemaphoreType.DMA,
            pltpu.SemaphoreType.DMA,
            pltpu.SemaphoreType.DMA,
            pltpu.SemaphoreType.DMA,
            pltpu.SemaphoreType.DMA,
            pltpu.SemaphoreType.DMA,
            pltpu.SemaphoreType.DMA,
        ],
    )
    def k(hl_hbm, hr_hbm, epk_hbm, wa_hbm, zeros_hbm, out_hbm,
          epk, hlr, hrr, ctb, dstb, attb, wav, acc,
          semi0, semi1, semgl0, semgl1, semgr0, semgr1, semsc0, semsc1):
        semi = (semi0, semi1)
        semgl = (semgl0, semgl1)
        semgr = (semgr0, semgr1)
        semsc = (semsc0, semsc1)
        cid = lax.axis_index("c")
        sid = lax.axis_index("s")
        pltpu.sync_copy(wa_hbm, wav)

        @pl.when(sid == 0)
        def _zero():
            pltpu.sync_copy(zeros_hbm, acc)

        plsc.subcore_barrier()

        b0 = (cid * NS + sid) * NBLK
        wev = [wav[0, pl.ds(16 * f, 16)] for f in range(8)]
        attv = [wav[1, pl.ds(16 * f, 16)] for f in range(8)]
        lanes = lax.iota(jnp.int32, 16)
        den_idx = H + (lanes & 7)
        den_msk = lanes < 8

        def idx_start(bi, s):
            return pltpu.async_copy(epk_hbm.at[b0 + bi], epk.at[s], semi[s])

        def idx_wait(s):
            pltpu.make_async_copy(epk_hbm.at[0], epk.at[s], semi[s]).wait()

        def g_start(s):
            pltpu.async_copy(hl_hbm.at[epk.at[s, 0]], hlr.at[s], semgl[s])
            pltpu.async_copy(hr_hbm.at[epk.at[s, 1]], hrr.at[s], semgr[s])

        def g_wait(s):
            pltpu.make_async_copy(
                hl_hbm.at[epk.at[s, 0]], hlr.at[s], semgl[s]).wait()
            pltpu.make_async_copy(
                hr_hbm.at[epk.at[s, 1]], hrr.at[s], semgr[s]).wait()

        def sc_start(s):
            pltpu.async_copy(ctb.at[s], acc.at[dstb.at[s]], semsc[s], add=True)

        def sc_wait(s):
            pltpu.make_async_copy(ctb.at[s], acc.at[dstb.at[s]], semsc[s]).wait()

        def compute(B, s):
            for g in range(BLK // 16):
                e0 = g * 16
                av = plsc.bitcast(attb[s, pl.ds(e0, 16)], jnp.float32)
                for j in range(16):
                    e = e0 + j
                    a = av[j]
                    accv = jnp.zeros((16,), jnp.float32)
                    for f in range(8):
                        sl = pl.ds(16 * f, 16)
                        m = hlr[s, e, sl] + hrr[s, e, sl] + a * wev[f]
                        m = jnp.where(m >= 0.0, m, 0.2 * m)
                        accv = accv + m * attv[f]
                    ex = jnp.exp(jnp.full((16,), jnp.sum(accv), jnp.float32))
                    ex = jnp.where(B * BLK + e < EPW, ex, 0.0)
                    for f in range(8):
                        sl = pl.ds(16 * f, 16)
                        ctb[s, e, sl] = ex * hlr[s, e, sl]
                    plsc.store_scatter(ctb.at[s, e], [den_idx], ex, mask=den_msk)

        def sub_body(B, s):
            o = 1 - s

            @pl.when(B <= NBLK - 2)
            def _next():
                idx_wait(o)
                g_start(o)

            g_wait(s)

            @pl.when(B >= 2)
            def _drain():
                sc_wait(s)

            for v in range(BLK // 16):
                sl = pl.ds(16 * v, 16)
                dstb[s, sl] = epk[s, 1, sl]
                attb[s, sl] = epk[s, 2, sl]

            @pl.when(B <= NBLK - 3)
            def _prefetch():
                idx_start(B + 2, s)

            compute(B, s)
            sc_start(s)

        idx_start(0, 0)
        idx_start(1, 1)
        idx_wait(0)
        g_start(0)

        def pair_body(i, carry):
            sub_body(2 * i, 0)
            sub_body(2 * i + 1, 1)
            return carry

        lax.fori_loop(0, NBLK // 2, pair_body, 0)

        sc_wait(0)
        sc_wait(1)
        plsc.subcore_barrier()

        @pl.when(sid == 0)
        def _writeback():
            pltpu.sync_copy(acc, out_hbm.at[cid])

    return k(hl, hr, epack, wa, zeros)


_DN_NT = (((1,), (1,)), ((), ()))
_DN_NN = (((1,), (0,)), ((), ()))


def _dual_mm(h, Wl, Wr, bl, br):
    def body(h_ref, wl_ref, wr_ref, bl_ref, br_ref, hl_ref, hr_ref):
        hb = h_ref[...]
        hl_ref[...] = lax.dot_general(
            hb, wl_ref[...], _DN_NT, preferred_element_type=jnp.float32) + bl_ref[...]
        hr_ref[...] = lax.dot_general(
            hb, wr_ref[...], _DN_NT, preferred_element_type=jnp.float32) + br_ref[...]

    return pl.pallas_call(
        body,
        grid=(N // BN,),
        in_specs=[
            pl.BlockSpec((BN, H), lambda i: (i, 0)),
            pl.BlockSpec((H, H), lambda i: (0, 0)),
            pl.BlockSpec((H, H), lambda i: (0, 0)),
            pl.BlockSpec((1, H), lambda i: (0, 0)),
            pl.BlockSpec((1, H), lambda i: (0, 0)),
        ],
        out_specs=[pl.BlockSpec((BN, H), lambda i: (i, 0)),
                   pl.BlockSpec((BN, H), lambda i: (i, 0))],
        out_shape=[jax.ShapeDtypeStruct((N, H), jnp.float32),
                   jax.ShapeDtypeStruct((N, H), jnp.float32)],
    )(h, Wl, Wr, bl, br)


def _combine(acc_ref, bp_ref):
    num = acc_ref[0, :, :H] + acc_ref[1, :, :H]
    den = acc_ref[0, :, H:H + 1] + acc_ref[1, :, H:H + 1]
    return num / (den + 1e-16) + bp_ref[...]


def _combine_mm(acc, bprev, Wl, Wr, bl, br):
    def body(acc_ref, bp_ref, wl_ref, wr_ref, bl_ref, br_ref, hl_ref, hr_ref):
        hb = jnp.maximum(_combine(acc_ref, bp_ref), 0.0)
        hl_ref[...] = lax.dot_general(
            hb, wl_ref[...], _DN_NT, preferred_element_type=jnp.float32) + bl_ref[...]
        hr_ref[...] = lax.dot_general(
            hb, wr_ref[...], _DN_NT, preferred_element_type=jnp.float32) + br_ref[...]

    return pl.pallas_call(
        body,
        grid=(N // BN,),
        in_specs=[
            pl.BlockSpec((NC, BN, ACCW), lambda i: (0, i, 0)),
            pl.BlockSpec((1, H), lambda i: (0, 0)),
            pl.BlockSpec((H, H), lambda i: (0, 0)),
            pl.BlockSpec((H, H), lambda i: (0, 0)),
            pl.BlockSpec((1, H), lambda i: (0, 0)),
            pl.BlockSpec((1, H), lambda i: (0, 0)),
        ],
        out_specs=[pl.BlockSpec((BN, H), lambda i: (i, 0)),
                   pl.BlockSpec((BN, H), lambda i: (i, 0))],
        out_shape=[jax.ShapeDtypeStruct((N, H), jnp.float32),
                   jax.ShapeDtypeStruct((N, H), jnp.float32)],
    )(acc, bprev, Wl, Wr, bl, br)


def _final(acc, bprev, batch3d, Wlin, blin):
    nblk = N // BN

    def body(acc_ref, bp_ref, b_ref, wlin_ref, blin_ref, out_ref, sums, cnt):
        i = pl.program_id(0)

        @pl.when(i == 0)
        def _init():
            sums[...] = jnp.zeros_like(sums)
            cnt[...] = jnp.zeros_like(cnt)

        hb = _combine(acc_ref, bp_ref)
        bi = b_ref[0]
        gi = lax.broadcasted_iota(jnp.int32, (G, BN), 0)
        oh = (jnp.broadcast_to(bi, (G, BN)) == gi).astype(jnp.float32)
        sums[...] += lax.dot_general(oh, hb, _DN_NN,
                                     preferred_element_type=jnp.float32)
        cnt[...] += lax.dot_general(oh, jnp.ones((BN, H), jnp.float32), _DN_NN,
                                    preferred_element_type=jnp.float32)

        @pl.when(i == nblk - 1)
        def _emit():
            hG = sums[...] / jnp.maximum(cnt[...], 1.0)
            out_ref[...] = lax.dot_general(
                hG, wlin_ref[...], _DN_NT,
                preferred_element_type=jnp.float32) + blin_ref[...]

    return pl.pallas_call(
        body,
        grid=(nblk,),
        in_specs=[
            pl.BlockSpec((NC, BN, ACCW), lambda i: (0, i, 0)),
            pl.BlockSpec((1, H), lambda i: (0, 0)),
            pl.BlockSpec((1, 1, BN), lambda i: (i, 0, 0)),
            pl.BlockSpec((OUT, H), lambda i: (0, 0)),
            pl.BlockSpec((1, OUT), lambda i: (0, 0)),
        ],
        out_specs=pl.BlockSpec((G, OUT), lambda i: (0, 0)),
        out_shape=jax.ShapeDtypeStruct((G, OUT), jnp.float32),
        scratch_shapes=[pltpu.VMEM((G, H), jnp.float32),
                        pltpu.VMEM((G, H), jnp.float32)],
    )(acc, bprev, batch3d, Wlin, blin)


def kernel(x, edge_index, edge_attr, batch,
           Wl1, bl1, Wr1, br1, We1, att1, b1,
           Wl2, bl2, Wr2, br2, We2, att2, b2,
           Wl3, bl3, Wr3, br3, We3, att3, b3,
           Wlin, blin):
    attr_bits = lax.bitcast_convert_type(edge_attr[:, 0], jnp.int32)
    trip = jnp.stack([edge_index[0], edge_index[1], attr_bits])
    trip = jnp.pad(trip.reshape(3, NW, EPW),
                   ((0, 0), (0, 0), (0, NBLK * BLK - EPW)))
    epack = trip.reshape(3, NW * NBLK, BLK).transpose(1, 0, 2)
    zeros = jnp.zeros((N, ACCW), jnp.float32)

    def wa(We, att):
        return jnp.stack([We[:, 0], att])

    hl, hr = _dual_mm(x, Wl1, Wr1, bl1.reshape(1, H), br1.reshape(1, H))
    acc = _edge_pass(hl, hr, epack, wa(We1, att1), zeros)
    hl, hr = _combine_mm(acc, b1.reshape(1, H), Wl2, Wr2,
                         bl2.reshape(1, H), br2.reshape(1, H))
    acc = _edge_pass(hl, hr, epack, wa(We2, att2), zeros)
    hl, hr = _combine_mm(acc, b2.reshape(1, H), Wl3, Wr3,
                         bl3.reshape(1, H), br3.reshape(1, H))
    acc = _edge_pass(hl, hr, epack, wa(We3, att3), zeros)
    return _final(acc, b3.reshape(1, H), batch.reshape(N // BN, 1, BN),
                  Wlin, blin.reshape(1, OUT))


# speedup vs baseline: 1.8326x; 1.8326x over previous
import functools

import jax
import jax.numpy as jnp
from jax import lax
from jax.experimental import pallas as pl
from jax.experimental.pallas import tpu as pltpu
from jax.experimental.pallas import tpu_sc as plsc

N = 10000
E = 320000
H = 128
OUT = 64
G = 64

ACCW = 144
NC = 2
NS = 16
NW = NC * NS
EPW = E // NW
BLK = 48
NBLK = 210
BN = 1000


def _edge_pass(hl, hr, epack, wa, zeros):
    mesh = plsc.VectorSubcoreMesh(core_axis_name="c", subcore_axis_name="s")

    @functools.partial(
        pl.kernel,
        mesh=mesh,
        compiler_params=pltpu.CompilerParams(
            needs_layout_passes=False, use_tc_tiling_on_sc=False),
        out_type=jax.ShapeDtypeStruct((NC, N, ACCW), jnp.float32),
        scratch_types=[
            pltpu.VMEM((2, 3, BLK), jnp.int32),
            pltpu.VMEM((2, BLK, H), jnp.float32),
            pltpu.VMEM((2, BLK, H), jnp.float32),
            pltpu.VMEM((BLK, ACCW), jnp.float32),
            pltpu.VMEM((BLK,), jnp.int32),
            pltpu.VMEM((BLK,), jnp.int32),
            pltpu.VMEM((2, H), jnp.float32),
            pltpu.VMEM_SHARED((N, ACCW), jnp.float32),
            pltpu.SemaphoreType.DMA,
            pltpu.SemaphoreType.DMA,
            pltpu.SemaphoreType.DMA,
            pltpu.SemaphoreType.DMA,
            pltpu.SemaphoreType.DMA,
            pltpu.SemaphoreType.DMA,
            pltpu.SemaphoreType.DMA,
        ],
    )
    def k(hl_hbm, hr_hbm, epk_hbm, wa_hbm, zeros_hbm, out_hbm,
          epk, hlr, hrr, ctb, dstb, attb, wav, acc,
          semi0, semi1, semgl0, semgl1, semgr0, semgr1, semsc):
        semi = (semi0, semi1)
        semgl = (semgl0, semgl1)
        semgr = (semgr0, semgr1)
        cid = lax.axis_index("c")
        sid = lax.axis_index("s")
        pltpu.sync_copy(wa_hbm, wav)

        @pl.when(sid == 0)
        def _zero():
            pltpu.sync_copy(zeros_hbm, acc)

        plsc.subcore_barrier()

        b0 = (cid * NS + sid) * NBLK
        wev = [wav[0, pl.ds(16 * f, 16)] for f in range(8)]
        attv = [wav[1, pl.ds(16 * f, 16)] for f in range(8)]

        def idx_start(bi, s):
            return pltpu.async_copy(epk_hbm.at[b0 + bi], epk.at[s], semi[s])

        def idx_wait(s):
            pltpu.make_async_copy(epk_hbm.at[0], epk.at[s], semi[s]).wait()

        def g_start(s):
            pltpu.async_copy(hl_hbm.at[epk.at[s, 0]], hlr.at[s], semgl[s])
            pltpu.async_copy(hr_hbm.at[epk.at[s, 1]], hrr.at[s], semgr[s])

        def g_wait(s):
            pltpu.make_async_copy(
                hl_hbm.at[epk.at[s, 0]], hlr.at[s], semgl[s]).wait()
            pltpu.make_async_copy(
                hr_hbm.at[epk.at[s, 1]], hrr.at[s], semgr[s]).wait()

        def sc_start():
            pltpu.async_copy(ctb, acc.at[dstb], semsc, add=True)

        def sc_wait():
            pltpu.make_async_copy(ctb, acc.at[dstb], semsc).wait()

        def compute(B, s):
            for g in range(BLK // 16):
                e0 = g * 16
                av = plsc.bitcast(attb[pl.ds(e0, 16)], jnp.float32)
                for j in range(16):
                    e = e0 + j
                    a = av[j]
                    accv = jnp.zeros((16,), jnp.float32)
                    for f in range(8):
                        sl = pl.ds(16 * f, 16)
                        m = hlr[s, e, sl] + hrr[s, e, sl] + a * wev[f]
                        m = jnp.where(m >= 0.0, m, 0.2 * m)
                        accv = accv + m * attv[f]
                    ex = jnp.exp(jnp.full((16,), jnp.sum(accv), jnp.float32))
                    ex = jnp.where(B * BLK + e < EPW, ex, 0.0)
                    for f in range(8):
                        sl = pl.ds(16 * f, 16)
                        ctb[e, sl] = ex * hlr[s, e, sl]
                    ctb[e, pl.ds(H, 16)] = ex

        def sub_body(B, s):
            o = 1 - s

            @pl.when(B <= NBLK - 2)
            def _next():
                idx_wait(o)
                g_start(o)

            g_wait(s)

            @pl.when(B >= 1)
            def _drain():
                sc_wait()

            for v in range(BLK // 16):
                sl = pl.ds(16 * v, 16)
                dstb[sl] = epk[s, 1, sl]
                attb[sl] = epk[s, 2, sl]

            @pl.when(B <= NBLK - 3)
            def _prefetch():
                idx_start(B + 2, s)

            compute(B, s)
            sc_start()

        idx_start(0, 0)
        idx_start(1, 1)
        idx_wait(0)
        g_start(0)

        def pair_body(i, carry):
            sub_body(2 * i, 0)
            sub_body(2 * i + 1, 1)
            return carry

        lax.fori_loop(0, NBLK // 2, pair_body, 0)

        sc_wait()
        plsc.subcore_barrier()

        @pl.when(sid == 0)
        def _writeback():
            pltpu.sync_copy(acc, out_hbm.at[cid])

    return k(hl, hr, epack, wa, zeros)


_DN_NT = (((1,), (1,)), ((), ()))
_DN_NN = (((1,), (0,)), ((), ()))


def _dual_mm(h, Wl, Wr, bl, br):
    def body(h_ref, wl_ref, wr_ref, bl_ref, br_ref, hl_ref, hr_ref):
        hb = h_ref[...]
        hl_ref[...] = lax.dot_general(
            hb, wl_ref[...], _DN_NT, preferred_element_type=jnp.float32) + bl_ref[...]
        hr_ref[...] = lax.dot_general(
            hb, wr_ref[...], _DN_NT, preferred_element_type=jnp.float32) + br_ref[...]

    return pl.pallas_call(
        body,
        grid=(N // BN,),
        in_specs=[
            pl.BlockSpec((BN, H), lambda i: (i, 0)),
            pl.BlockSpec((H, H), lambda i: (0, 0)),
            pl.BlockSpec((H, H), lambda i: (0, 0)),
            pl.BlockSpec((1, H), lambda i: (0, 0)),
            pl.BlockSpec((1, H), lambda i: (0, 0)),
        ],
        out_specs=[pl.BlockSpec((BN, H), lambda i: (i, 0)),
                   pl.BlockSpec((BN, H), lambda i: (i, 0))],
        out_shape=[jax.ShapeDtypeStruct((N, H), jnp.float32),
                   jax.ShapeDtypeStruct((N, H), jnp.float32)],
    )(h, Wl, Wr, bl, br)


def _combine(acc_ref, bp_ref):
    num = acc_ref[0, :, :H] + acc_ref[1, :, :H]
    den = acc_ref[0, :, H:H + 1] + acc_ref[1, :, H:H + 1]
    return num / (den + 1e-16) + bp_ref[...]


def _combine_mm(acc, bprev, Wl, Wr, bl, br):
    def body(acc_ref, bp_ref, wl_ref, wr_ref, bl_ref, br_ref, hl_ref, hr_ref):
        hb = jnp.maximum(_combine(acc_ref, bp_ref), 0.0)
        hl_ref[...] = lax.dot_general(
            hb, wl_ref[...], _DN_NT, preferred_element_type=jnp.float32) + bl_ref[...]
        hr_ref[...] = lax.dot_general(
            hb, wr_ref[...], _DN_NT, preferred_element_type=jnp.float32) + br_ref[...]

    return pl.pallas_call(
        body,
        grid=(N // BN,),
        in_specs=[
            pl.BlockSpec((NC, BN, ACCW), lambda i: (0, i, 0)),
            pl.BlockSpec((1, H), lambda i: (0, 0)),
            pl.BlockSpec((H, H), lambda i: (0, 0)),
            pl.BlockSpec((H, H), lambda i: (0, 0)),
            pl.BlockSpec((1, H), lambda i: (0, 0)),
            pl.BlockSpec((1, H), lambda i: (0, 0)),
        ],
        out_specs=[pl.BlockSpec((BN, H), lambda i: (i, 0)),
                   pl.BlockSpec((BN, H), lambda i: (i, 0))],
        out_shape=[jax.ShapeDtypeStruct((N, H), jnp.float32),
                   jax.ShapeDtypeStruct((N, H), jnp.float32)],
    )(acc, bprev, Wl, Wr, bl, br)


def _final(acc, bprev, batch3d, Wlin, blin):
    nblk = N // BN

    def body(acc_ref, bp_ref, b_ref, wlin_ref, blin_ref, out_ref, sums, cnt):
        i = pl.program_id(0)

        @pl.when(i == 0)
        def _init():
            sums[...] = jnp.zeros_like(sums)
            cnt[...] = jnp.zeros_like(cnt)

        hb = _combine(acc_ref, bp_ref)
        bi = b_ref[0]
        gi = lax.broadcasted_iota(jnp.int32, (G, BN), 0)
        oh = (jnp.broadcast_to(bi, (G, BN)) == gi).astype(jnp.float32)
        sums[...] += lax.dot_general(oh, hb, _DN_NN,
                                     preferred_element_type=jnp.float32)
        cnt[...] += lax.dot_general(oh, jnp.ones((BN, H), jnp.float32), _DN_NN,
                                    preferred_element_type=jnp.float32)

        @pl.when(i == nblk - 1)
        def _emit():
            hG = sums[...] / jnp.maximum(cnt[...], 1.0)
            out_ref[...] = lax.dot_general(
                hG, wlin_ref[...], _DN_NT,
                preferred_element_type=jnp.float32) + blin_ref[...]

    return pl.pallas_call(
        body,
        grid=(nblk,),
        in_specs=[
            pl.BlockSpec((NC, BN, ACCW), lambda i: (0, i, 0)),
            pl.BlockSpec((1, H), lambda i: (0, 0)),
            pl.BlockSpec((1, 1, BN), lambda i: (i, 0, 0)),
            pl.BlockSpec((OUT, H), lambda i: (0, 0)),
            pl.BlockSpec((1, OUT), lambda i: (0, 0)),
        ],
        out_specs=pl.BlockSpec((G, OUT), lambda i: (0, 0)),
        out_shape=jax.ShapeDtypeStruct((G, OUT), jnp.float32),
        scratch_shapes=[pltpu.VMEM((G, H), jnp.float32),
                        pltpu.VMEM((G, H), jnp.float32)],
    )(acc, bprev, batch3d, Wlin, blin)


def kernel(x, edge_index, edge_attr, batch,
           Wl1, bl1, Wr1, br1, We1, att1, b1,
           Wl2, bl2, Wr2, br2, We2, att2, b2,
           Wl3, bl3, Wr3, br3, We3, att3, b3,
           Wlin, blin):
    attr_bits = lax.bitcast_convert_type(edge_attr[:, 0], jnp.int32)
    trip = jnp.stack([edge_index[0], edge_index[1], attr_bits])
    trip = jnp.pad(trip.reshape(3, NW, EPW),
                   ((0, 0), (0, 0), (0, NBLK * BLK - EPW)))
    epack = trip.reshape(3, NW * NBLK, BLK).transpose(1, 0, 2)
    zeros = jnp.zeros((N, ACCW), jnp.float32)

    def wa(We, att):
        return jnp.stack([We[:, 0], att])

    hl, hr = _dual_mm(x, Wl1, Wr1, bl1.reshape(1, H), br1.reshape(1, H))
    acc = _edge_pass(hl, hr, epack, wa(We1, att1), zeros)
    hl, hr = _combine_mm(acc, b1.reshape(1, H), Wl2, Wr2,
                         bl2.reshape(1, H), br2.reshape(1, H))
    acc = _edge_pass(hl, hr, epack, wa(We2, att2), zeros)
    hl, hr = _combine_mm(acc, b2.reshape(1, H), Wl3, Wr3,
                         bl3.reshape(1, H), br3.reshape(1, H))
    acc = _edge_pass(hl, hr, epack, wa(We3, att3), zeros)
    return _final(acc, b3.reshape(1, H), batch.reshape(N // BN, 1, BN),
                  Wlin, blin.reshape(1, OUT))
